# Initial kernel scaffold; baseline (speedup 1.0000x reference)
import jax
import jax.numpy as jnp
from jax.experimental import pallas as pl


def kernel(x, adj_indices, adj_values, W1, b1, W2, b2, keep_rate):
    raise NotImplementedError("write your pallas kernel here")


# speedup vs baseline: 3.5038x; 3.5038x over previous
import functools

import jax
import jax.numpy as jnp
from jax import lax
from jax.experimental import pallas as pl
from jax.experimental.pallas import tpu as pltpu
from jax.experimental.pallas import tpu_sc as plsc

N = 10000
E = 320000
D = 128

NW = 32
C = 80
NCHUNK = 128
G = 4
NG = NCHUNK // G
EPW = C * NCHUNK
EPAD = NW * EPW

_mesh = plsc.VectorSubcoreMesh(
    core_axis_name="c", subcore_axis_name="s", num_cores=2, num_subcores=16
)

_DNUMS = lax.GatherDimensionNumbers(
    offset_dims=(), collapsed_slice_dims=(0,), start_index_map=(0,))


@functools.partial(
    pl.kernel,
    out_type=jax.ShapeDtypeStruct((2, N, D), jnp.float32),
    mesh=_mesh,
    scratch_types=[
        pltpu.VMEM((2, G, 2, C), jnp.int32),
        pltpu.VMEM((2, G, C), jnp.float32),
        pltpu.VMEM((G, C, D), jnp.float32),
        pltpu.VMEM_SHARED((N, D), jnp.float32),
        pltpu.SemaphoreType.DMA((G,)),
        pltpu.SemaphoreType.DMA((G,)),
        pltpu.SemaphoreType.DMA,
    ],
)
def _sc_aggregate(h_hbm, pk_hbm, val_hbm, out_hbm, ring, vring, rows, acc,
                  gsem, ssem, isem):
    c = lax.axis_index("c")
    s = lax.axis_index("s")
    wid = s * 2 + c

    def _zrow(r, carry):
        for j in range(D // 16):
            rows[0, r, pl.ds(16 * j, 16)] = jnp.zeros((16,), jnp.float32)
        return carry
    lax.fori_loop(0, C, _zrow, 0)
    base = s * 1000

    @pl.when(s < 10)
    def _zero_acc():
        for t in range(13):
            off = 920 if t == 12 else C * t
            pltpu.sync_copy(rows.at[0], acc.at[pl.ds(base + off, C)])
    plsc.subcore_barrier()

    def _mult(p, b):
        for g in range(C // 16):
            vv = vring[p, b, pl.ds(g * 16, 16)]

            def _lane(l, carry):
                lv = jnp.full((16,), 0, jnp.int32) + l
                vsplat = lax.gather(
                    vv, lv[:, None], _DNUMS, slice_sizes=(1,),
                    mode=lax.GatherScatterMode.PROMISE_IN_BOUNDS)
                e = g * 16 + l
                for j in range(D // 16):
                    sl = pl.ds(16 * j, 16)
                    rows[b, e, sl] = rows[b, e, sl] * vsplat
                return carry
            lax.fori_loop(0, 16, _lane, 0, unroll=2)

    pltpu.sync_copy(pk_hbm.at[wid].at[pl.ds(0, G)], ring.at[0])
    pltpu.sync_copy(val_hbm.at[wid].at[pl.ds(0, G)], vring.at[0])
    pltpu.async_copy(pk_hbm.at[wid].at[pl.ds(G, G)], ring.at[1], isem)
    pltpu.async_copy(val_hbm.at[wid].at[pl.ds(G, G)], vring.at[1], isem)
    for b in range(G):
        pltpu.async_copy(h_hbm.at[ring.at[0, b, 0]], rows.at[b], gsem.at[b])

    def _proc(p, m):
        q = 1 - p
        for b in range(G):
            pltpu.make_async_copy(h_hbm.at[ring.at[p, b, 0]], rows.at[b],
                                  gsem.at[b]).wait()
            _mult(p, b)
            pltpu.async_copy(rows.at[b], acc.at[ring.at[p, b, 1]],
                             ssem.at[b], add=True)

        @pl.when(m + 1 < NG)
        def _refill():
            pltpu.make_async_copy(pk_hbm.at[wid].at[pl.ds(0, G)],
                                  ring.at[q], isem).wait()
            pltpu.make_async_copy(val_hbm.at[wid].at[pl.ds(0, G)],
                                  vring.at[q], isem).wait()
            for b in range(G):
                pltpu.make_async_copy(rows.at[b], acc.at[ring.at[p, b, 1]],
                                      ssem.at[b]).wait()
                pltpu.async_copy(h_hbm.at[ring.at[q, b, 0]], rows.at[b],
                                 gsem.at[b])

            @pl.when(m + 2 < NG)
            def _prefetch():
                pltpu.async_copy(pk_hbm.at[wid].at[pl.ds((m + 2) * G, G)],
                                 ring.at[p], isem)
                pltpu.async_copy(val_hbm.at[wid].at[pl.ds((m + 2) * G, G)],
                                 vring.at[p], isem)

        @pl.when(m + 1 >= NG)
        def _last():
            for b in range(G):
                pltpu.make_async_copy(rows.at[b], acc.at[ring.at[p, b, 1]],
                                      ssem.at[b]).wait()

    def _pair(ml, carry):
        _proc(0, 2 * ml)
        _proc(1, 2 * ml + 1)
        return carry

    lax.fori_loop(0, NG // 2, _pair, 0)
    plsc.subcore_barrier()

    @pl.when(s < 10)
    def _publish():
        pltpu.sync_copy(acc.at[pl.ds(base, 1000)],
                        out_hbm.at[c].at[pl.ds(base, 1000)])


_BLK = 1000


def _linear_body(p_ref, w_ref, b_ref, o_ref):
    sm = p_ref[0] + p_ref[1]
    o_ref[...] = lax.dot_general(
        sm, w_ref[...], (((1,), (1,)), ((), ())),
        preferred_element_type=jnp.float32) + b_ref[...]


def _final_body(p_ref, w_ref, b_ref, x_ref, h1_ref, o_ref):
    sm = p_ref[0] + p_ref[1]
    h2 = lax.dot_general(
        sm, w_ref[...], (((1,), (1,)), ((), ())),
        preferred_element_type=jnp.float32) + b_ref[...]
    o_ref[...] = (x_ref[...] + h1_ref[...] + h2) * jnp.float32(1.0 / 3.0)


def _tc_linear(p, w, b):
    return pl.pallas_call(
        _linear_body,
        grid=(N // _BLK,),
        in_specs=[
            pl.BlockSpec((2, _BLK, D), lambda i: (0, i, 0)),
            pl.BlockSpec((D, D), lambda i: (0, 0)),
            pl.BlockSpec((1, D), lambda i: (0, 0)),
        ],
        out_specs=pl.BlockSpec((_BLK, D), lambda i: (i, 0)),
        out_shape=jax.ShapeDtypeStruct((N, D), jnp.float32),
    )(p, w, b.reshape(1, D))


def _tc_final(p, w, b, x, h1):
    return pl.pallas_call(
        _final_body,
        grid=(N // _BLK,),
        in_specs=[
            pl.BlockSpec((2, _BLK, D), lambda i: (0, i, 0)),
            pl.BlockSpec((D, D), lambda i: (0, 0)),
            pl.BlockSpec((1, D), lambda i: (0, 0)),
            pl.BlockSpec((_BLK, D), lambda i: (i, 0)),
            pl.BlockSpec((_BLK, D), lambda i: (i, 0)),
        ],
        out_specs=pl.BlockSpec((_BLK, D), lambda i: (i, 0)),
        out_shape=jax.ShapeDtypeStruct((N, D), jnp.float32),
    )(p, w, b.reshape(1, D), x, h1)


def kernel(x, adj_indices, adj_values, W1, b1, W2, b2, keep_rate):
    row = adj_indices[0].astype(jnp.int32)
    col = adj_indices[1].astype(jnp.int32)
    vals = adj_values.astype(jnp.float32) * jnp.asarray(keep_rate, jnp.float32)

    pad = EPAD - E
    zi = jnp.zeros((pad,), jnp.int32)
    row3 = jnp.concatenate([row, zi]).reshape(NW, NCHUNK, C)
    col3 = jnp.concatenate([col, zi]).reshape(NW, NCHUNK, C)
    vals3 = jnp.concatenate([vals, jnp.zeros((pad,), jnp.float32)]).reshape(
        NW, NCHUNK, C)
    pk = jnp.stack([col3, row3], axis=2)

    p1 = _sc_aggregate(x, pk, vals3)
    h1 = _tc_linear(p1, W1, b1)
    p2 = _sc_aggregate(h1, pk, vals3)
    return _tc_final(p2, W2, b2, x, h1)
